# Initial kernel scaffold; baseline (speedup 1.0000x reference)
import functools

import jax
import jax.numpy as jnp
from jax import lax
from jax.experimental import pallas as pl
from jax.experimental.pallas import tpu as pltpu
from jax.experimental.pallas import tpu_sc as plsc

_KS = 16


def _knn_body(sup_ref, src_ref, idx_ref, *, K, N, TM):
    b = pl.program_id(0)
    s = sup_ref[0]
    p = src_ref[0]
    sp = jnp.dot(s, p, preferred_element_type=jnp.float32)
    pn = jnp.sum(p * p, axis=0, keepdims=True)
    sn = jnp.sum(s * s, axis=1, keepdims=True)
    d2 = pn - 2.0 * sp + sn
    iota = lax.broadcasted_iota(jnp.int32, (TM, N), 1)
    cols = []
    for _ in range(K):
        m = jnp.min(d2, axis=1, keepdims=True)
        cand = jnp.where(d2 == m, iota, N)
        a = jnp.min(cand, axis=1, keepdims=True)
        cols.append(a)
        d2 = jnp.where(iota == a, jnp.inf, d2)
    idx_ref[0] = jnp.concatenate(cols, axis=1) + b * N


def _knn(sup, src, K):
    B, M, _ = sup.shape
    N = src.shape[2]
    TM = min(M, 128)
    body = functools.partial(_knn_body, K=K, N=N, TM=TM)
    return pl.pallas_call(
        body,
        grid=(B, M // TM),
        in_specs=[
            pl.BlockSpec((1, TM, 3), lambda b, i: (b, i, 0)),
            pl.BlockSpec((1, 3, N), lambda b, i: (b, 0, 0)),
        ],
        out_specs=pl.BlockSpec((1, TM, K), lambda b, i: (b, i, 0)),
        out_shape=jax.ShapeDtypeStruct((B, M, K), jnp.int32),
    )(sup, src)


def _sc_gather(table, idx_flat):
    R, D = table.shape
    Btot = idx_flat.shape[0]
    NW = 32
    b_per_w = Btot // NW
    assert b_per_w * NW == Btot and b_per_w % 8 == 0
    CH = min(128, b_per_w)
    n = b_per_w // CH
    g = max(1, min(n, (384 * 1024) // (CH * D * 4), 16))
    while n % g:
        g -= 1
    ngroups = n // g
    idx2 = idx_flat.reshape(Btot // CH, CH)

    mesh = plsc.VectorSubcoreMesh(core_axis_name="c", subcore_axis_name="s")

    @functools.partial(
        pl.kernel,
        mesh=mesh,
        out_type=jax.ShapeDtypeStruct((Btot, D), jnp.float32),
        scratch_types=[
            pltpu.VMEM((g, CH), jnp.int32),
            pltpu.VMEM((g * CH, D), jnp.float32),
            pltpu.SemaphoreType.DMA,
        ],
    )
    def k(table_hbm, idx_hbm, out_hbm, idx_v, rows_v, sem):
        wid = lax.axis_index("s") * 2 + lax.axis_index("c")
        chunk0 = wid * n

        def group(grp, _):
            cbase = chunk0 + grp * g
            pltpu.sync_copy(idx_hbm.at[pl.ds(cbase, g)], idx_v)
            copies = []
            for j in range(g):
                copies.append(pltpu.async_copy(
                    table_hbm.at[idx_v.at[j]],
                    rows_v.at[pl.ds(j * CH, CH)], sem))
            for c in copies:
                c.wait()
            pltpu.sync_copy(rows_v, out_hbm.at[pl.ds(cbase * CH, g * CH)])
            return 0

        if ngroups == 1:
            group(0, 0)
        else:
            lax.fori_loop(0, ngroups, group, 0)

    return k(table, idx2)


def _conv_body(gp_ref, gf_ref, sup_ref, w1_ref, b1_ref, w2_ref, b2_ref,
               w3_ref, b3_ref, W_ref, bo_ref, out_ref, *, K, C, TP):
    s = sup_ref[:, :3]
    rels = []
    dists = []
    for k in range(K):
        npk = gp_ref[k][:, :3]
        rel = npk - s
        d2 = jnp.sum(rel * rel, axis=1, keepdims=True)
        dists.append(jnp.sqrt(d2 + 1e-9))
        rels.append(rel)
    maxd = functools.reduce(jnp.maximum, dists)
    inv = 1.0 / (maxd + 1e-9)
    rel_all = jnp.concatenate(rels, axis=0) * jnp.concatenate([inv] * K, 0)
    h = jnp.dot(rel_all, w1_ref[:], preferred_element_type=jnp.float32)
    h = jax.nn.relu(h + b1_ref[:])
    h = jnp.dot(h, w2_ref[:], preferred_element_type=jnp.float32)
    h = jax.nn.relu(h + b2_ref[:])
    w = jnp.dot(h, w3_ref[:], preferred_element_type=jnp.float32) + b3_ref[:]
    fs = [jnp.zeros((TP, C), jnp.float32) for _ in range(_KS)]
    for k in range(K):
        wk = w[k * TP:(k + 1) * TP]
        nf = gf_ref[k][:, :C]
        for si in range(_KS):
            fs[si] = fs[si] + wk[:, si:si + 1] * nf
    fsc = jnp.concatenate(fs, axis=1)
    out = jnp.dot(fsc, W_ref[:], preferred_element_type=jnp.float32)
    out_ref[:] = out * (1.0 / K) + bo_ref[:]


def _conv_core(gp, gf, sup, w1, b1, w2, b2, w3, b3, W, bo, K, C, Cout):
    P = sup.shape[0]
    Cp = gf.shape[2]
    TP = min(P, 256)
    body = functools.partial(_conv_body, K=K, C=C, TP=TP)
    return pl.pallas_call(
        body,
        grid=(P // TP,),
        in_specs=[
            pl.BlockSpec((K, TP, 16), lambda i: (0, i, 0)),
            pl.BlockSpec((K, TP, Cp), lambda i: (0, i, 0)),
            pl.BlockSpec((TP, 16), lambda i: (i, 0)),
            pl.BlockSpec(w1.shape, lambda i: (0, 0)),
            pl.BlockSpec(b1.shape, lambda i: (0, 0)),
            pl.BlockSpec(w2.shape, lambda i: (0, 0)),
            pl.BlockSpec(b2.shape, lambda i: (0, 0)),
            pl.BlockSpec(w3.shape, lambda i: (0, 0)),
            pl.BlockSpec(b3.shape, lambda i: (0, 0)),
            pl.BlockSpec(W.shape, lambda i: (0, 0)),
            pl.BlockSpec(bo.shape, lambda i: (0, 0)),
        ],
        out_specs=pl.BlockSpec((TP, Cout), lambda i: (i, 0)),
        out_shape=jax.ShapeDtypeStruct((P, Cout), jnp.float32),
    )(gp, gf, sup, w1, b1, w2, b2, w3, b3, W, bo)


def _bn_body(y_ref, g_ref, b_ref, out_ref):
    y = y_ref[:]
    mean = jnp.mean(y, axis=0, keepdims=True)
    c = y - mean
    var = jnp.mean(c * c, axis=0, keepdims=True)
    out = c * lax.rsqrt(var + 1e-5) * g_ref[:] + b_ref[:]
    out_ref[:] = jax.nn.relu(out)


def _bn_act(y, gamma, beta):
    P, C = y.shape
    return pl.pallas_call(
        _bn_body,
        grid=(1,),
        in_specs=[
            pl.BlockSpec((P, C), lambda i: (0, 0)),
            pl.BlockSpec((1, C), lambda i: (0, 0)),
            pl.BlockSpec((1, C), lambda i: (0, 0)),
        ],
        out_specs=pl.BlockSpec((P, C), lambda i: (0, 0)),
        out_shape=jax.ShapeDtypeStruct((P, C), jnp.float32),
    )(y, gamma.reshape(1, C), beta.reshape(1, C))


def _fc_body(y_ref, w_ref, b_ref, out_ref):
    out_ref[:] = jnp.dot(y_ref[:], w_ref[:],
                         preferred_element_type=jnp.float32) + b_ref[:]


def _fc(y, w, b):
    P, C = y.shape
    O = w.shape[1]
    TP = 2048
    return pl.pallas_call(
        _fc_body,
        grid=(P // TP,),
        in_specs=[
            pl.BlockSpec((TP, C), lambda i: (i, 0)),
            pl.BlockSpec((C, O), lambda i: (0, 0)),
            pl.BlockSpec((1, O), lambda i: (0, 0)),
        ],
        out_specs=pl.BlockSpec((TP, O), lambda i: (i, 0)),
        out_shape=jax.ShapeDtypeStruct((P, O), jnp.float32),
    )(y, w, b.reshape(1, O))


def _pad16(a):
    c = a.shape[1]
    cp = (c + 15) // 16 * 16
    if cp == c:
        return a
    return jnp.pad(a, ((0, 0), (0, cp - c)))


def _conv_layer(p, feat, ptsT, pts_bpn, Nsrc, M, K, bn, act):
    B = ptsT.shape[0]
    C = feat.shape[1]
    sup3 = ptsT[:, :M, :]
    idx = _knn(sup3, pts_bpn[:, :, :Nsrc], K)
    P = B * M
    idx_flat = idx.transpose(2, 0, 1).reshape(K * P)
    pts_rows = _pad16(ptsT[:, :Nsrc, :].reshape(B * Nsrc, 3))
    gp = _sc_gather(pts_rows, idx_flat).reshape(K, P, 16)
    gf = _sc_gather(_pad16(feat), idx_flat)
    gf = gf.reshape(K, P, gf.shape[1])
    sup_rows = _pad16(sup3.reshape(P, 3))
    w1 = p['w1']
    w1c = w1.reshape(_KS, 3, w1.shape[1]).sum(0)
    b1p = (p['b1'] - p['centers'].reshape(-1) @ w1).reshape(1, -1)
    out = _conv_core(
        gp, gf, sup_rows,
        w1c, b1p,
        p['w2'], p['b2'].reshape(1, -1),
        p['w3'], p['b3'].reshape(1, -1),
        p['W'], p['b'].reshape(1, -1),
        K, C, p['W'].shape[1])
    if bn:
        out = _bn_act(out, p['gamma'], p['beta'])
    return out


def kernel(x, input_pts, params):
    B, CIN, N = x.shape
    ptsT = input_pts.transpose(0, 2, 1)
    feat0 = x.transpose(0, 2, 1).reshape(B * N, CIN)

    def conv(name, feat, Nsrc, M, K, bn=True, act=True):
        return _conv_layer(params[name], feat, ptsT, input_pts,
                           Nsrc, M, K, bn, act)

    x0 = conv('cv0', feat0, 4096, 4096, 16, bn=False, act=False)
    x1 = conv('cv1', x0, 4096, 2048, 16)
    x2 = conv('cv2', x1, 2048, 1024, 16)
    x3 = conv('cv3', x2, 1024, 256, 16)
    x4 = conv('cv4', x3, 256, 64, 16)
    x5 = conv('cv5', x4, 64, 16, 16)
    x6 = conv('cv6', x5, 16, 8, 16)
    x5d = jnp.concatenate([conv('cv5d', x6, 8, 16, 4), x5], axis=1)
    x4d = jnp.concatenate([conv('cv4d', x5d, 16, 64, 4), x4], axis=1)
    x3d = jnp.concatenate([conv('cv3d', x4d, 64, 256, 4), x3], axis=1)
    x2d = jnp.concatenate([conv('cv2d', x3d, 256, 1024, 8), x2], axis=1)
    x1d = jnp.concatenate([conv('cv1d', x2d, 1024, 2048, 8), x1], axis=1)
    x0d = jnp.concatenate([conv('cv0d', x1d, 2048, 4096, 8), x0], axis=1)
    out = _fc(x0d, params['fcout_w'].T, params['fcout_b'])
    return out.reshape(B, N, -1).transpose(0, 2, 1)


# speedup vs baseline: 8.6209x; 8.6209x over previous
import functools

import jax
import jax.numpy as jnp
from jax import lax
from jax.experimental import pallas as pl
from jax.experimental.pallas import tpu as pltpu
from jax.experimental.pallas import tpu_sc as plsc

_KS = 16


def _knn_body(sup_ref, src_ref, idx_ref, *, K, N, TM):
    b = pl.program_id(0)
    s = sup_ref[0]
    p = src_ref[0]
    sp = jnp.dot(s, p, preferred_element_type=jnp.float32)
    pn = jnp.sum(p * p, axis=0, keepdims=True)
    sn = jnp.sum(s * s, axis=1, keepdims=True)
    d2 = pn - 2.0 * sp + sn
    iota = lax.broadcasted_iota(jnp.int32, (TM, N), 1)
    cols = []
    for _ in range(K):
        m = jnp.min(d2, axis=1, keepdims=True)
        cand = jnp.where(d2 == m, iota, N)
        a = jnp.min(cand, axis=1, keepdims=True)
        cols.append(a)
        d2 = jnp.where(iota == a, jnp.inf, d2)
    idx_ref[0] = jnp.concatenate(cols, axis=1) + b * N


def _knn(sup, src, K):
    B, M, _ = sup.shape
    N = src.shape[2]
    TM = min(M, 128)
    body = functools.partial(_knn_body, K=K, N=N, TM=TM)
    return pl.pallas_call(
        body,
        grid=(B, M // TM),
        in_specs=[
            pl.BlockSpec((1, TM, 3), lambda b, i: (b, i, 0)),
            pl.BlockSpec((1, 3, N), lambda b, i: (b, 0, 0)),
        ],
        out_specs=pl.BlockSpec((1, TM, K), lambda b, i: (b, i, 0)),
        out_shape=jax.ShapeDtypeStruct((B, M, K), jnp.int32),
    )(sup, src)


def _sc_gather(table, idx_flat):
    R, D = table.shape
    Btot = idx_flat.shape[0]
    NW = 32
    b_per_w = Btot // NW
    assert b_per_w * NW == Btot and b_per_w % 8 == 0
    CH = min(128, b_per_w)
    n = b_per_w // CH
    g = max(1, min(n, (384 * 1024) // (CH * D * 4), 16))
    while n % g:
        g -= 1
    ngroups = n // g
    idx2 = idx_flat.reshape(Btot // CH, CH)

    mesh = plsc.VectorSubcoreMesh(core_axis_name="c", subcore_axis_name="s")

    @functools.partial(
        pl.kernel,
        mesh=mesh,
        out_type=jax.ShapeDtypeStruct((Btot, D), jnp.float32),
        scratch_types=[
            pltpu.VMEM((g, CH), jnp.int32),
            pltpu.VMEM((g * CH, D), jnp.float32),
            pltpu.SemaphoreType.DMA,
        ],
        compiler_params=pltpu.CompilerParams(use_tc_tiling_on_sc=False),
    )
    def k(table_hbm, idx_hbm, out_hbm, idx_v, rows_v, sem):
        wid = lax.axis_index("s") * 2 + lax.axis_index("c")
        chunk0 = wid * n

        def group(grp, _):
            cbase = chunk0 + grp * g
            pltpu.sync_copy(idx_hbm.at[pl.ds(cbase, g)], idx_v)
            copies = []
            for j in range(g):
                copies.append(pltpu.async_copy(
                    table_hbm.at[idx_v.at[j]],
                    rows_v.at[pl.ds(j * CH, CH)], sem))
            for c in copies:
                c.wait()
            pltpu.sync_copy(rows_v, out_hbm.at[pl.ds(cbase * CH, g * CH)])
            return 0

        if ngroups == 1:
            group(0, 0)
        else:
            lax.fori_loop(0, ngroups, group, 0)

    return k(table, idx2)


def _bdot(a, b):
    return jnp.dot(a.astype(jnp.bfloat16), b.astype(jnp.bfloat16),
                   preferred_element_type=jnp.float32)


def _conv_body(gp_ref, gf_ref, sup_ref, w1_ref, b1_ref, w2_ref, b2_ref,
               w3_ref, b3_ref, W_ref, bo_ref, cf_ref, out_ref, *, K, C, TP):
    s = sup_ref[:, :3]
    rels = []
    dists = []
    for k in range(K):
        npk = gp_ref[k][:, :3]
        rel = npk - s
        d2 = jnp.sum(rel * rel, axis=1, keepdims=True)
        dists.append(jnp.sqrt(d2 + 1e-9))
        rels.append(rel)
    maxd = functools.reduce(jnp.maximum, dists)
    scale = maxd + 1e-9
    rel_all = jnp.concatenate(rels, axis=0) / jnp.concatenate([scale] * K, 0)
    d = jnp.concatenate([rel_all] * _KS, axis=1) - cf_ref[:]
    h = jax.nn.relu(_bdot(d, w1_ref[:]) + b1_ref[:])
    h = jax.nn.relu(_bdot(h, w2_ref[:]) + b2_ref[:])
    w = _bdot(h, w3_ref[:]) + b3_ref[:]
    fs = [jnp.zeros((TP, C), jnp.float32) for _ in range(_KS)]
    for k in range(K):
        wk = w[k * TP:(k + 1) * TP].astype(jnp.bfloat16).astype(jnp.float32)
        nf = gf_ref[k][:, :C].astype(jnp.bfloat16).astype(jnp.float32)
        for si in range(_KS):
            fs[si] = fs[si] + wk[:, si:si + 1] * nf
    fsc = jnp.concatenate(fs, axis=1) * (1.0 / K)
    out_ref[:] = _bdot(fsc, W_ref[:]) + bo_ref[:]


def _conv_core(gp, gf, sup, w1, b1, w2, b2, w3, b3, W, bo, cf, K, C, Cout):
    P = sup.shape[0]
    Cp = gf.shape[2]
    TP = min(P, 256)
    body = functools.partial(_conv_body, K=K, C=C, TP=TP)
    full = lambda a: pl.BlockSpec(a.shape, lambda i: tuple(0 for _ in a.shape))
    return pl.pallas_call(
        body,
        grid=(P // TP,),
        in_specs=[
            pl.BlockSpec((K, TP, 16), lambda i: (0, i, 0)),
            pl.BlockSpec((K, TP, Cp), lambda i: (0, i, 0)),
            pl.BlockSpec((TP, 16), lambda i: (i, 0)),
            full(w1), full(b1), full(w2), full(b2), full(w3), full(b3),
            full(W), full(bo), full(cf),
        ],
        out_specs=pl.BlockSpec((TP, Cout), lambda i: (i, 0)),
        out_shape=jax.ShapeDtypeStruct((P, Cout), jnp.float32),
    )(gp, gf, sup, w1, b1, w2, b2, w3, b3, W, bo, cf)


def _bn_body(y_ref, g_ref, b_ref, out_ref):
    y = y_ref[:]
    mean = jnp.mean(y, axis=0, keepdims=True)
    c = y - mean
    var = jnp.mean(c * c, axis=0, keepdims=True)
    out = c * lax.rsqrt(var + 1e-5) * g_ref[:] + b_ref[:]
    out_ref[:] = jax.nn.relu(out)


def _bn_act(y, gamma, beta):
    P, C = y.shape
    return pl.pallas_call(
        _bn_body,
        grid=(1,),
        in_specs=[
            pl.BlockSpec((P, C), lambda i: (0, 0)),
            pl.BlockSpec((1, C), lambda i: (0, 0)),
            pl.BlockSpec((1, C), lambda i: (0, 0)),
        ],
        out_specs=pl.BlockSpec((P, C), lambda i: (0, 0)),
        out_shape=jax.ShapeDtypeStruct((P, C), jnp.float32),
    )(y, gamma.reshape(1, C), beta.reshape(1, C))


def _fc_body(y_ref, w_ref, b_ref, out_ref):
    out_ref[:] = _bdot(y_ref[:], w_ref[:]) + b_ref[:]


def _fc(y, w, b):
    P, C = y.shape
    O = w.shape[1]
    TP = 2048
    return pl.pallas_call(
        _fc_body,
        grid=(P // TP,),
        in_specs=[
            pl.BlockSpec((TP, C), lambda i: (i, 0)),
            pl.BlockSpec((C, O), lambda i: (0, 0)),
            pl.BlockSpec((1, O), lambda i: (0, 0)),
        ],
        out_specs=pl.BlockSpec((TP, O), lambda i: (i, 0)),
        out_shape=jax.ShapeDtypeStruct((P, O), jnp.float32),
    )(y, w, b.reshape(1, O))


def _pad16(a):
    c = a.shape[1]
    cp = (c + 15) // 16 * 16
    if cp == c:
        return a
    return jnp.pad(a, ((0, 0), (0, cp - c)))


def _conv_layer(p, feat, ptsT, pts_bpn, Nsrc, M, K, bn, act):
    B = ptsT.shape[0]
    C = feat.shape[1]
    sup3 = ptsT[:, :M, :]
    idx = _knn(sup3, pts_bpn[:, :, :Nsrc], K)
    P = B * M
    idx_flat = idx.transpose(2, 0, 1).reshape(K * P)
    pts_rows = _pad16(ptsT[:, :Nsrc, :].reshape(B * Nsrc, 3))
    gp = _sc_gather(pts_rows, idx_flat).reshape(K, P, 16)
    gf = _sc_gather(_pad16(feat), idx_flat)
    gf = gf.reshape(K, P, gf.shape[1])
    sup_rows = _pad16(sup3.reshape(P, 3))
    cf = p['centers'].reshape(1, 3 * _KS)
    out = _conv_core(
        gp, gf, sup_rows,
        p['w1'], p['b1'].reshape(1, -1),
        p['w2'], p['b2'].reshape(1, -1),
        p['w3'], p['b3'].reshape(1, -1),
        p['W'], p['b'].reshape(1, -1),
        cf, K, C, p['W'].shape[1])
    if bn:
        out = _bn_act(out, p['gamma'], p['beta'])
    return out


def kernel(x, input_pts, params):
    B, CIN, N = x.shape
    ptsT = input_pts.transpose(0, 2, 1)
    feat0 = x.transpose(0, 2, 1).reshape(B * N, CIN)

    def conv(name, feat, Nsrc, M, K, bn=True, act=True):
        return _conv_layer(params[name], feat, ptsT, input_pts,
                           Nsrc, M, K, bn, act)

    x0 = conv('cv0', feat0, 4096, 4096, 16, bn=False, act=False)
    x1 = conv('cv1', x0, 4096, 2048, 16)
    x2 = conv('cv2', x1, 2048, 1024, 16)
    x3 = conv('cv3', x2, 1024, 256, 16)
    x4 = conv('cv4', x3, 256, 64, 16)
    x5 = conv('cv5', x4, 64, 16, 16)
    x6 = conv('cv6', x5, 16, 8, 16)
    x5d = jnp.concatenate([conv('cv5d', x6, 8, 16, 4), x5], axis=1)
    x4d = jnp.concatenate([conv('cv4d', x5d, 16, 64, 4), x4], axis=1)
    x3d = jnp.concatenate([conv('cv3d', x4d, 64, 256, 4), x3], axis=1)
    x2d = jnp.concatenate([conv('cv2d', x3d, 256, 1024, 8), x2], axis=1)
    x1d = jnp.concatenate([conv('cv1d', x2d, 1024, 2048, 8), x1], axis=1)
    x0d = jnp.concatenate([conv('cv0d', x1d, 2048, 4096, 8), x0], axis=1)
    out = _fc(x0d, params['fcout_w'].T, params['fcout_b'])
    return out.reshape(B, N, -1).transpose(0, 2, 1)


# speedup vs baseline: 9.8822x; 1.1463x over previous
import functools

import jax
import jax.numpy as jnp
from jax import lax
from jax.experimental import pallas as pl
from jax.experimental.pallas import tpu as pltpu
from jax.experimental.pallas import tpu_sc as plsc

_KS = 16


def _knn_body(sup_ref, src_ref, idx_ref, *, K, N, TM):
    b = pl.program_id(0)
    s = sup_ref[0]
    p = src_ref[0]
    sp = jnp.dot(s, p, preferred_element_type=jnp.float32)
    pn = jnp.sum(p * p, axis=0, keepdims=True)
    sn = jnp.sum(s * s, axis=1, keepdims=True)
    d2 = pn - 2.0 * sp + sn
    iota = lax.broadcasted_iota(jnp.int32, (TM, N), 1).astype(jnp.float32)
    cols = []
    for _ in range(K):
        m = jnp.min(d2, axis=1, keepdims=True)
        cand = jnp.where(d2 == m, iota, jnp.float32(N))
        a = jnp.min(cand, axis=1, keepdims=True)
        cols.append(a)
        d2 = jnp.where(iota == a, jnp.inf, d2)
    idxf = jnp.concatenate(cols, axis=1)
    idx_ref[0] = idxf.astype(jnp.int32) + b * N


def _knn(sup, src, K):
    B, M, _ = sup.shape
    N = src.shape[2]
    TM = min(M, 128)
    body = functools.partial(_knn_body, K=K, N=N, TM=TM)
    return pl.pallas_call(
        body,
        grid=(B, M // TM),
        in_specs=[
            pl.BlockSpec((1, TM, 3), lambda b, i: (b, i, 0)),
            pl.BlockSpec((1, 3, N), lambda b, i: (b, 0, 0)),
        ],
        out_specs=pl.BlockSpec((1, TM, K), lambda b, i: (b, i, 0)),
        out_shape=jax.ShapeDtypeStruct((B, M, K), jnp.int32),
    )(sup, src)


def _sc_gather2(tab_a, tab_b, idx_flat):
    Da = tab_a.shape[1]
    Db = tab_b.shape[1]
    Btot = idx_flat.shape[0]
    NW = 32
    b_per_w = Btot // NW
    assert b_per_w * NW == Btot and b_per_w % 8 == 0
    CH = min(128, b_per_w)
    n = b_per_w // CH
    g = max(1, min(n, (320 * 1024) // (CH * (Da + Db) * 4), 8))
    while n % g:
        g -= 1
    ngroups = n // g
    idx2 = idx_flat.reshape(Btot // CH, CH)

    mesh = plsc.VectorSubcoreMesh(core_axis_name="c", subcore_axis_name="s")

    @functools.partial(
        pl.kernel,
        mesh=mesh,
        out_type=(jax.ShapeDtypeStruct((Btot, Da), jnp.float32),
                  jax.ShapeDtypeStruct((Btot, Db), jnp.float32)),
        scratch_types=[
            pltpu.VMEM((g, CH), jnp.int32),
            pltpu.VMEM((g * CH, Da), jnp.float32),
            pltpu.VMEM((g * CH, Db), jnp.float32),
            pltpu.SemaphoreType.DMA,
        ],
        compiler_params=pltpu.CompilerParams(use_tc_tiling_on_sc=False),
    )
    def k(ta_hbm, tb_hbm, idx_hbm, oa_hbm, ob_hbm, idx_v, ra_v, rb_v, sem):
        wid = lax.axis_index("s") * 2 + lax.axis_index("c")
        chunk0 = wid * n

        def group(grp, _):
            cbase = chunk0 + grp * g
            pltpu.sync_copy(idx_hbm.at[pl.ds(cbase, g)], idx_v)
            copies = []
            for j in range(g):
                copies.append(pltpu.async_copy(
                    ta_hbm.at[idx_v.at[j]],
                    ra_v.at[pl.ds(j * CH, CH)], sem))
                copies.append(pltpu.async_copy(
                    tb_hbm.at[idx_v.at[j]],
                    rb_v.at[pl.ds(j * CH, CH)], sem))
            for c in copies:
                c.wait()
            pltpu.sync_copy(ra_v, oa_hbm.at[pl.ds(cbase * CH, g * CH)])
            pltpu.sync_copy(rb_v, ob_hbm.at[pl.ds(cbase * CH, g * CH)])
            return 0

        if ngroups == 1:
            group(0, 0)
        else:
            lax.fori_loop(0, ngroups, group, 0)

    return k(tab_a, tab_b, idx2)


def _bdot(a, b):
    return jnp.dot(a.astype(jnp.bfloat16), b.astype(jnp.bfloat16),
                   preferred_element_type=jnp.float32)


def _conv_body(gp_ref, gf_ref, sup_ref, w1_ref, b1_ref, w2_ref, b2_ref,
               w3_ref, b3_ref, W_ref, bo_ref, cf_ref, out_ref, *, K, C, TP):
    s = sup_ref[:, :3]
    rels = []
    dists = []
    for k in range(K):
        npk = gp_ref[k][:, :3]
        rel = npk - s
        d2 = jnp.sum(rel * rel, axis=1, keepdims=True)
        dists.append(jnp.sqrt(d2 + 1e-9))
        rels.append(rel)
    maxd = functools.reduce(jnp.maximum, dists)
    scale = maxd + 1e-9
    rel_all = jnp.concatenate(rels, axis=0) / jnp.concatenate([scale] * K, 0)
    d = jnp.concatenate([rel_all] * _KS, axis=1) - cf_ref[:]
    h = jax.nn.relu(_bdot(d, w1_ref[:]) + b1_ref[:])
    h = jax.nn.relu(_bdot(h, w2_ref[:]) + b2_ref[:])
    w = _bdot(h, w3_ref[:]) + b3_ref[:]
    fs = [jnp.zeros((TP, C), jnp.float32) for _ in range(_KS)]
    for k in range(K):
        wk = w[k * TP:(k + 1) * TP].astype(jnp.bfloat16).astype(jnp.float32)
        nf = gf_ref[k][:, :C].astype(jnp.bfloat16).astype(jnp.float32)
        for si in range(_KS):
            fs[si] = fs[si] + wk[:, si:si + 1] * nf
    fsc = jnp.concatenate(fs, axis=1) * (1.0 / K)
    out_ref[:] = _bdot(fsc, W_ref[:]) + bo_ref[:]


def _conv_core(gp, gf, sup, w1, b1, w2, b2, w3, b3, W, bo, cf, K, C, Cout):
    P = sup.shape[0]
    Cp = gf.shape[2]
    TP = min(P, 256)
    body = functools.partial(_conv_body, K=K, C=C, TP=TP)
    full = lambda a: pl.BlockSpec(a.shape, lambda i: tuple(0 for _ in a.shape))
    return pl.pallas_call(
        body,
        grid=(P // TP,),
        in_specs=[
            pl.BlockSpec((K, TP, 16), lambda i: (0, i, 0)),
            pl.BlockSpec((K, TP, Cp), lambda i: (0, i, 0)),
            pl.BlockSpec((TP, 16), lambda i: (i, 0)),
            full(w1), full(b1), full(w2), full(b2), full(w3), full(b3),
            full(W), full(bo), full(cf),
        ],
        out_specs=pl.BlockSpec((TP, Cout), lambda i: (i, 0)),
        out_shape=jax.ShapeDtypeStruct((P, Cout), jnp.float32),
    )(gp, gf, sup, w1, b1, w2, b2, w3, b3, W, bo, cf)


def _bn_body(y_ref, g_ref, b_ref, out_ref):
    y = y_ref[:]
    mean = jnp.mean(y, axis=0, keepdims=True)
    c = y - mean
    var = jnp.mean(c * c, axis=0, keepdims=True)
    out = c * lax.rsqrt(var + 1e-5) * g_ref[:] + b_ref[:]
    out_ref[:] = jax.nn.relu(out)


def _bn_act(y, gamma, beta):
    P, C = y.shape
    return pl.pallas_call(
        _bn_body,
        grid=(1,),
        in_specs=[
            pl.BlockSpec((P, C), lambda i: (0, 0)),
            pl.BlockSpec((1, C), lambda i: (0, 0)),
            pl.BlockSpec((1, C), lambda i: (0, 0)),
        ],
        out_specs=pl.BlockSpec((P, C), lambda i: (0, 0)),
        out_shape=jax.ShapeDtypeStruct((P, C), jnp.float32),
    )(y, gamma.reshape(1, C), beta.reshape(1, C))


def _fc_body(y_ref, w_ref, b_ref, out_ref):
    out_ref[:] = _bdot(y_ref[:], w_ref[:]) + b_ref[:]


def _fc(y, w, b):
    P, C = y.shape
    O = w.shape[1]
    TP = 2048
    return pl.pallas_call(
        _fc_body,
        grid=(P // TP,),
        in_specs=[
            pl.BlockSpec((TP, C), lambda i: (i, 0)),
            pl.BlockSpec((C, O), lambda i: (0, 0)),
            pl.BlockSpec((1, O), lambda i: (0, 0)),
        ],
        out_specs=pl.BlockSpec((TP, O), lambda i: (i, 0)),
        out_shape=jax.ShapeDtypeStruct((P, O), jnp.float32),
    )(y, w, b.reshape(1, O))


def _pad16(a):
    c = a.shape[1]
    cp = (c + 15) // 16 * 16
    if cp == c:
        return a
    return jnp.pad(a, ((0, 0), (0, cp - c)))


def _conv_layer(p, feat, ptsT, pts_bpn, Nsrc, M, K, bn, act):
    B = ptsT.shape[0]
    C = feat.shape[1]
    sup3 = ptsT[:, :M, :]
    idx = _knn(sup3, pts_bpn[:, :, :Nsrc], K)
    P = B * M
    idx_flat = idx.transpose(2, 0, 1).reshape(K * P)
    pts_rows = _pad16(ptsT[:, :Nsrc, :].reshape(B * Nsrc, 3))
    gp, gf = _sc_gather2(pts_rows, _pad16(feat), idx_flat)
    gp = gp.reshape(K, P, 16)
    gf = gf.reshape(K, P, gf.shape[1])
    sup_rows = _pad16(sup3.reshape(P, 3))
    cf = p['centers'].reshape(1, 3 * _KS)
    out = _conv_core(
        gp, gf, sup_rows,
        p['w1'], p['b1'].reshape(1, -1),
        p['w2'], p['b2'].reshape(1, -1),
        p['w3'], p['b3'].reshape(1, -1),
        p['W'], p['b'].reshape(1, -1),
        cf, K, C, p['W'].shape[1])
    if bn:
        out = _bn_act(out, p['gamma'], p['beta'])
    return out


def kernel(x, input_pts, params):
    B, CIN, N = x.shape
    ptsT = input_pts.transpose(0, 2, 1)
    feat0 = x.transpose(0, 2, 1).reshape(B * N, CIN)

    def conv(name, feat, Nsrc, M, K, bn=True, act=True):
        return _conv_layer(params[name], feat, ptsT, input_pts,
                           Nsrc, M, K, bn, act)

    x0 = conv('cv0', feat0, 4096, 4096, 16, bn=False, act=False)
    x1 = conv('cv1', x0, 4096, 2048, 16)
    x2 = conv('cv2', x1, 2048, 1024, 16)
    x3 = conv('cv3', x2, 1024, 256, 16)
    x4 = conv('cv4', x3, 256, 64, 16)
    x5 = conv('cv5', x4, 64, 16, 16)
    x6 = conv('cv6', x5, 16, 8, 16)
    x5d = jnp.concatenate([conv('cv5d', x6, 8, 16, 4), x5], axis=1)
    x4d = jnp.concatenate([conv('cv4d', x5d, 16, 64, 4), x4], axis=1)
    x3d = jnp.concatenate([conv('cv3d', x4d, 64, 256, 4), x3], axis=1)
    x2d = jnp.concatenate([conv('cv2d', x3d, 256, 1024, 8), x2], axis=1)
    x1d = jnp.concatenate([conv('cv1d', x2d, 1024, 2048, 8), x1], axis=1)
    x0d = jnp.concatenate([conv('cv0d', x1d, 2048, 4096, 8), x0], axis=1)
    out = _fc(x0d, params['fcout_w'].T, params['fcout_b'])
    return out.reshape(B, N, -1).transpose(0, 2, 1)
